# baseline (speedup 1.0000x reference)
import functools

import jax
import jax.numpy as jnp
import numpy as np
from jax import lax
from jax.experimental import pallas as pl
from jax.experimental.pallas import tpu as pltpu
from jax.experimental.pallas import tpu_sc as plsc

N = 32768
D = 768
H = 64
O = 16
E = 8
PER = N // E

TILE = 1024
NC = 2
NS = 16
NW = NC * NS
CHUNK = N // NW
K = CHUNK // 128


def _sc_route_eid(idx3, eid3):
    mesh = plsc.VectorSubcoreMesh(core_axis_name="core", subcore_axis_name="subcore")

    @functools.partial(
        pl.kernel,
        out_type=jax.ShapeDtypeStruct((N,), jnp.int32),
        mesh=mesh,
        scratch_types=[
            pltpu.VMEM((K, 128), jnp.int32),
            pltpu.VMEM((K, 128), jnp.int32),
        ],
    )
    def sc_kernel(idx_hbm, eid_hbm, out_hbm, idx_v, val_v):
        wid = lax.axis_index("subcore") * NC + lax.axis_index("core")
        pltpu.sync_copy(idx_hbm.at[wid], idx_v)
        pltpu.sync_copy(eid_hbm.at[wid], val_v)
        for j in range(K):
            pltpu.sync_copy(val_v.at[j], out_hbm.at[idx_v.at[j]])

    return sc_kernel(idx3, eid3)


def _mlp_kernel(emb_ref, eid_ref, w1_ref, b1_ref, w2_ref, b2_ref, out_ref):
    x = emb_ref[...].astype(jnp.bfloat16)
    h = jax.lax.dot_general(
        x, w1_ref[...], (((1,), (0,)), ((), ())),
        preferred_element_type=jnp.float32,
    ) + b1_ref[...]
    h = h * jax.nn.sigmoid(h)
    o_all = jax.lax.dot_general(
        h.astype(jnp.bfloat16), w2_ref[...], (((1,), (0,)), ((), ())),
        preferred_element_type=jnp.float32,
    ) + b2_ref[...]
    lane_e = jax.lax.broadcasted_iota(jnp.int32, (TILE, E * O), 1) // O
    om = jnp.where(eid_ref[...] == lane_e, o_all, 0.0)
    acc = om[:, 0:O]
    for e in range(1, E):
        acc = acc + om[:, e * O:(e + 1) * O]
    out_ref[...] = acc


def kernel(species, embedding, idx_0, idx_1, idx_2, idx_3, idx_4, idx_5,
           idx_6, idx_7, W1, b1, W2, b2):
    idx_cat = jnp.concatenate(
        [idx_0, idx_1, idx_2, idx_3, idx_4, idx_5, idx_6, idx_7])
    idx3 = idx_cat.astype(jnp.int32).reshape(NW, K, 128)
    eid3 = jnp.broadcast_to(
        jnp.arange(E, dtype=jnp.int32)[:, None], (E, PER)
    ).reshape(NW, K, 128)

    eid = _sc_route_eid(idx3, eid3).reshape(N, 1)

    w1_cat = jnp.transpose(W1, (1, 0, 2)).reshape(D, E * H).astype(jnp.bfloat16)
    b1_cat = b1.reshape(1, E * H)
    w2_big = jnp.zeros((E, H, E, O), jnp.float32)
    w2_big = w2_big.at[jnp.arange(E), :, jnp.arange(E), :].set(W2)
    w2_big = w2_big.reshape(E * H, E * O).astype(jnp.bfloat16)
    b2_big = b2.reshape(1, E * O)

    out = pl.pallas_call(
        _mlp_kernel,
        grid=(N // TILE,),
        in_specs=[
            pl.BlockSpec((TILE, D), lambda i: (i, 0)),
            pl.BlockSpec((TILE, 1), lambda i: (i, 0)),
            pl.BlockSpec((D, E * H), lambda i: (0, 0)),
            pl.BlockSpec((1, E * H), lambda i: (0, 0)),
            pl.BlockSpec((E * H, E * O), lambda i: (0, 0)),
            pl.BlockSpec((1, E * O), lambda i: (0, 0)),
        ],
        out_specs=pl.BlockSpec((TILE, O), lambda i: (i, 0)),
        out_shape=jax.ShapeDtypeStruct((N, O), jnp.float32),
        compiler_params=pltpu.CompilerParams(
            dimension_semantics=("arbitrary",),
        ),
    )(embedding, eid, w1_cat, b1_cat, w2_big, b2_big)
    return out


# speedup vs baseline: 1.0000x; 1.0000x over previous
import functools

import jax
import jax.numpy as jnp
import numpy as np
from jax import lax
from jax.experimental import pallas as pl
from jax.experimental.pallas import tpu as pltpu
from jax.experimental.pallas import tpu_sc as plsc

N = 32768
D = 768
H = 64
O = 16
E = 8
PER = N // E

TILE = 1024
NC = 2
NS = 16
NW = NC * NS
CHUNK = N // NW
K = CHUNK // 128


def _sc_route_eid(idx3, eid3):
    mesh = plsc.VectorSubcoreMesh(core_axis_name="core", subcore_axis_name="subcore")

    @functools.partial(
        pl.kernel,
        out_type=jax.ShapeDtypeStruct((N,), jnp.int32),
        mesh=mesh,
        scratch_types=[
            pltpu.VMEM((K, 128), jnp.int32),
            pltpu.VMEM((K, 128), jnp.int32),
        ],
    )
    def sc_kernel(idx_hbm, eid_hbm, out_hbm, idx_v, val_v):
        wid = lax.axis_index("subcore") * NC + lax.axis_index("core")
        pltpu.sync_copy(idx_hbm.at[wid], idx_v)
        pltpu.sync_copy(eid_hbm.at[wid], val_v)
        for j in range(K):
            pltpu.sync_copy(val_v.at[j], out_hbm.at[idx_v.at[j]])

    return sc_kernel(idx3, eid3)


def _mlp_kernel(emb_ref, eid_ref, w1_ref, b1_ref, w2_ref, b2_ref, out_ref):
    x = emb_ref[...].astype(jnp.bfloat16)
    h = jax.lax.dot_general(
        x, w1_ref[...], (((1,), (0,)), ((), ())),
        preferred_element_type=jnp.float32,
    ) + b1_ref[...]
    h = h * jax.nn.sigmoid(h)
    o_all = jax.lax.dot_general(
        h.astype(jnp.bfloat16), w2_ref[...], (((1,), (0,)), ((), ())),
        preferred_element_type=jnp.float32,
    ) + b2_ref[...]
    lane_e = jax.lax.broadcasted_iota(jnp.int32, (TILE, E * O), 1) // O
    om = jnp.where(eid_ref[...] == lane_e, o_all, 0.0)
    acc = om[:, 0:O]
    for e in range(1, E):
        acc = acc + om[:, e * O:(e + 1) * O]
    out_ref[...] = acc


def kernel(species, embedding, idx_0, idx_1, idx_2, idx_3, idx_4, idx_5,
           idx_6, idx_7, W1, b1, W2, b2):
    idx_cat = jnp.concatenate(
        [idx_0, idx_1, idx_2, idx_3, idx_4, idx_5, idx_6, idx_7])
    idx3 = idx_cat.astype(jnp.int32).reshape(NW, K, 128)
    eid3 = jnp.broadcast_to(
        jnp.arange(E, dtype=jnp.int32)[:, None], (E, PER)
    ).reshape(NW, K, 128)

    eid = _sc_route_eid(idx3, eid3).reshape(N, 1)

    w1_cat = jnp.transpose(W1, (1, 0, 2)).reshape(D, E * H).astype(jnp.bfloat16)
    b1_cat = b1.reshape(1, E * H)
    w2_big = jnp.zeros((E, H, E, O), jnp.float32)
    w2_big = w2_big.at[jnp.arange(E), :, jnp.arange(E), :].set(W2)
    w2_big = w2_big.reshape(E * H, E * O).astype(jnp.bfloat16)
    b2_big = b2.reshape(1, E * O)

    out = pl.pallas_call(
        _mlp_kernel,
        grid=(N // TILE,),
        in_specs=[
            pl.BlockSpec((TILE, D), lambda i: (i, 0)),
            pl.BlockSpec((TILE, 1), lambda i: (i, 0)),
            pl.BlockSpec((D, E * H), lambda i: (0, 0)),
            pl.BlockSpec((1, E * H), lambda i: (0, 0)),
            pl.BlockSpec((E * H, E * O), lambda i: (0, 0)),
            pl.BlockSpec((1, E * O), lambda i: (0, 0)),
        ],
        out_specs=pl.BlockSpec((TILE, O), lambda i: (i, 0)),
        out_shape=jax.ShapeDtypeStruct((N, O), jnp.float32),
        compiler_params=pltpu.CompilerParams(
            dimension_semantics=("parallel",),
        ),
    )(embedding, eid, w1_cat, b1_cat, w2_big, b2_big)
    return out


# speedup vs baseline: 1.4705x; 1.4705x over previous
import functools

import jax
import jax.numpy as jnp
import numpy as np
from jax.experimental import pallas as pl
from jax.experimental.pallas import tpu as pltpu
from jax.experimental.pallas import tpu_sc as plsc

N = 32768
D = 768
H = 64
O = 16
E = 8
PER = N // E

TILE = 2048
SC_WIN = 256


def _sc_route_mask(onehot_src, idx_cat):
    idx2 = idx_cat.reshape(1, N)
    steps_per_block = PER // SC_WIN
    mesh = plsc.VectorSubcoreMesh(core_axis_name="core", subcore_axis_name="subcore")

    @functools.partial(
        pl.kernel,
        out_type=jax.ShapeDtypeStruct((N, 128), jnp.float32),
        mesh=mesh,
    )
    def sc_kernel(src_hbm, i_hbm, o_hbm):
        def body(src_vmem, i_vmem):
            pltpu.sync_copy(src_vmem, o_hbm.at[i_vmem.at[0]])

        pltpu.emit_pipeline(
            body,
            grid=(N // SC_WIN,),
            in_specs=[
                pl.BlockSpec((SC_WIN, 128), lambda i: (i // steps_per_block, 0)),
                pl.BlockSpec((1, SC_WIN), lambda i: (0, i)),
            ],
            out_specs=[],
            core_axis_name=("core", "subcore"),
            dimension_semantics=(pltpu.PARALLEL,),
        )(src_hbm, i_hbm)

    return sc_kernel(onehot_src, idx2)


def _mlp_kernel(emb_ref, mask_ref, w1_ref, b1_ref, w2_ref, b2_ref, out_ref):
    x = emb_ref[...].astype(jnp.bfloat16)
    h = jax.lax.dot_general(
        x, w1_ref[...], (((1,), (0,)), ((), ())),
        preferred_element_type=jnp.float32,
    ) + b1_ref[...]
    h = h * jax.nn.sigmoid(h)
    o_all = jax.lax.dot_general(
        h.astype(jnp.bfloat16), w2_ref[...], (((1,), (0,)), ((), ())),
        preferred_element_type=jnp.float32,
    ) + b2_ref[...]
    mrep = mask_ref[...].astype(jnp.float32)
    om = o_all * mrep
    acc = om[:, 0:O]
    for e in range(1, E):
        acc = acc + om[:, e * O:(e + 1) * O]
    out_ref[...] = acc


def kernel(species, embedding, idx_0, idx_1, idx_2, idx_3, idx_4, idx_5,
           idx_6, idx_7, W1, b1, W2, b2):
    idx_cat = jnp.concatenate(
        [idx_0, idx_1, idx_2, idx_3, idx_4, idx_5, idx_6, idx_7])

    onehot_src = jnp.broadcast_to(
        jnp.repeat(jnp.eye(E, dtype=jnp.float32), 16, axis=1)[:, None, :],
        (E, SC_WIN, 128),
    ).reshape(E * SC_WIN, 128)

    mask = _sc_route_mask(onehot_src, idx_cat)

    w1_cat = jnp.transpose(W1, (1, 0, 2)).reshape(D, E * H).astype(jnp.bfloat16)
    b1_cat = b1.reshape(1, E * H)
    w2_big = jnp.zeros((E, H, E, O), jnp.float32)
    w2_big = w2_big.at[jnp.arange(E), :, jnp.arange(E), :].set(W2)
    w2_big = w2_big.reshape(E * H, E * O).astype(jnp.bfloat16)
    b2_big = b2.reshape(1, E * O)

    out = pl.pallas_call(
        _mlp_kernel,
        grid=(N // TILE,),
        in_specs=[
            pl.BlockSpec((TILE, D), lambda i: (i, 0)),
            pl.BlockSpec((TILE, 128), lambda i: (i, 0)),
            pl.BlockSpec((D, E * H), lambda i: (0, 0)),
            pl.BlockSpec((1, E * H), lambda i: (0, 0)),
            pl.BlockSpec((E * H, E * O), lambda i: (0, 0)),
            pl.BlockSpec((1, E * O), lambda i: (0, 0)),
        ],
        out_specs=pl.BlockSpec((TILE, O), lambda i: (i, 0)),
        out_shape=jax.ShapeDtypeStruct((N, O), jnp.float32),
        compiler_params=pltpu.CompilerParams(
            dimension_semantics=("arbitrary",),
        ),
    )(embedding, mask, w1_cat, b1_cat, w2_big, b2_big)
    return out


# speedup vs baseline: 1.4868x; 1.0111x over previous
import functools

import jax
import jax.numpy as jnp
import numpy as np
from jax.experimental import pallas as pl
from jax.experimental.pallas import tpu as pltpu
from jax.experimental.pallas import tpu_sc as plsc

N = 32768
D = 768
H = 64
O = 16
E = 8
PER = N // E

TILE = 4096
SC_WIN = 256


def _sc_route_mask(onehot_src, idx_cat):
    idx2 = idx_cat.reshape(1, N)
    steps_per_block = PER // SC_WIN
    mesh = plsc.VectorSubcoreMesh(core_axis_name="core", subcore_axis_name="subcore")

    @functools.partial(
        pl.kernel,
        out_type=jax.ShapeDtypeStruct((N, 128), jnp.float32),
        mesh=mesh,
    )
    def sc_kernel(src_hbm, i_hbm, o_hbm):
        def body(src_vmem, i_vmem):
            pltpu.sync_copy(src_vmem, o_hbm.at[i_vmem.at[0]])

        pltpu.emit_pipeline(
            body,
            grid=(N // SC_WIN,),
            in_specs=[
                pl.BlockSpec((SC_WIN, 128), lambda i: (i // steps_per_block, 0)),
                pl.BlockSpec((1, SC_WIN), lambda i: (0, i)),
            ],
            out_specs=[],
            core_axis_name=("core", "subcore"),
            dimension_semantics=(pltpu.PARALLEL,),
        )(src_hbm, i_hbm)

    return sc_kernel(onehot_src, idx2)


def _mlp_kernel(emb_ref, mask_ref, w1_ref, b1_ref, w2_ref, b2_ref, out_ref):
    x = emb_ref[...].astype(jnp.bfloat16)
    h = jax.lax.dot_general(
        x, w1_ref[...], (((1,), (0,)), ((), ())),
        preferred_element_type=jnp.float32,
    ) + b1_ref[...]
    h = h * jax.nn.sigmoid(h)
    o_all = jax.lax.dot_general(
        h.astype(jnp.bfloat16), w2_ref[...], (((1,), (0,)), ((), ())),
        preferred_element_type=jnp.float32,
    ) + b2_ref[...]
    mrep = mask_ref[...].astype(jnp.float32)
    om = o_all * mrep
    acc = om[:, 0:O]
    for e in range(1, E):
        acc = acc + om[:, e * O:(e + 1) * O]
    out_ref[...] = acc


def kernel(species, embedding, idx_0, idx_1, idx_2, idx_3, idx_4, idx_5,
           idx_6, idx_7, W1, b1, W2, b2):
    idx_cat = jnp.concatenate(
        [idx_0, idx_1, idx_2, idx_3, idx_4, idx_5, idx_6, idx_7])

    onehot_src = jnp.broadcast_to(
        jnp.repeat(jnp.eye(E, dtype=jnp.float32), 16, axis=1)[:, None, :],
        (E, SC_WIN, 128),
    ).reshape(E * SC_WIN, 128)

    mask = _sc_route_mask(onehot_src, idx_cat)

    w1_cat = jnp.transpose(W1, (1, 0, 2)).reshape(D, E * H).astype(jnp.bfloat16)
    b1_cat = b1.reshape(1, E * H)
    w2_big = jnp.zeros((E, H, E, O), jnp.float32)
    w2_big = w2_big.at[jnp.arange(E), :, jnp.arange(E), :].set(W2)
    w2_big = w2_big.reshape(E * H, E * O).astype(jnp.bfloat16)
    b2_big = b2.reshape(1, E * O)

    out = pl.pallas_call(
        _mlp_kernel,
        grid=(N // TILE,),
        in_specs=[
            pl.BlockSpec((TILE, D), lambda i: (i, 0)),
            pl.BlockSpec((TILE, 128), lambda i: (i, 0)),
            pl.BlockSpec((D, E * H), lambda i: (0, 0)),
            pl.BlockSpec((1, E * H), lambda i: (0, 0)),
            pl.BlockSpec((E * H, E * O), lambda i: (0, 0)),
            pl.BlockSpec((1, E * O), lambda i: (0, 0)),
        ],
        out_specs=pl.BlockSpec((TILE, O), lambda i: (i, 0)),
        out_shape=jax.ShapeDtypeStruct((N, O), jnp.float32),
        compiler_params=pltpu.CompilerParams(
            dimension_semantics=("arbitrary",),
        ),
    )(embedding, mask, w1_cat, b1_cat, w2_big, b2_big)
    return out


# speedup vs baseline: 1.8324x; 1.2324x over previous
import functools

import jax
import jax.numpy as jnp
import numpy as np
from jax import lax
from jax.experimental import pallas as pl
from jax.experimental.pallas import tpu as pltpu
from jax.experimental.pallas import tpu_sc as plsc

N = 32768
D = 768
H = 64
O = 16
E = 8
PER = N // E

TILE = 4096
SC_NS = 16
SC_CHUNK = N // SC_NS
SC_K = SC_CHUNK // 128


def _sc_route_eid(idx3, eid3):
    mesh = plsc.VectorSubcoreMesh(
        core_axis_name="core", subcore_axis_name="subcore", num_cores=1)

    @functools.partial(
        pl.kernel,
        out_type=jax.ShapeDtypeStruct((N,), jnp.int32),
        mesh=mesh,
        scratch_types=[
            pltpu.VMEM((SC_K, 128), jnp.int32),
            pltpu.VMEM((SC_K, 128), jnp.int32),
            pltpu.VMEM((SC_CHUNK,), jnp.int32),
            pltpu.VMEM_SHARED((N,), jnp.int32),
        ],
    )
    def sc_kernel(idx_hbm, eid_hbm, out_hbm, idx_v, val_v, stage_v, shared):
        sid = lax.axis_index("subcore")
        base = sid * SC_CHUNK
        pltpu.sync_copy(idx_hbm.at[sid], idx_v)
        pltpu.sync_copy(eid_hbm.at[sid], val_v)
        for j in range(SC_K):
            pltpu.sync_copy(val_v.at[j], shared.at[idx_v.at[j]])
        plsc.subcore_barrier()
        pltpu.sync_copy(shared.at[pl.ds(base, SC_CHUNK)], stage_v)
        pltpu.sync_copy(stage_v, out_hbm.at[pl.ds(base, SC_CHUNK)])

    return sc_kernel(idx3, eid3)


def _mlp_kernel(emb_ref, eid_ref, w1_ref, b1_ref, w2_ref, out_ref):
    x = emb_ref[...].astype(jnp.bfloat16)
    h = jax.lax.dot_general(
        w1_ref[...], x, (((0,), (1,)), ((), ())),
        preferred_element_type=jnp.float32,
    ) + b1_ref[...]
    h = h * jax.nn.sigmoid(h)
    eid = eid_ref[...]
    slot = jax.lax.broadcasted_iota(jnp.int32, (E * H, 1), 0) // H
    hm = jnp.where(eid == slot, h, 0.0).astype(jnp.bfloat16)
    oh8 = jnp.where(
        eid == jax.lax.broadcasted_iota(jnp.int32, (E, 1), 0),
        jnp.float32(1.0), jnp.float32(0.0)).astype(jnp.bfloat16)
    hm_aug = jnp.concatenate([hm, oh8], axis=0)
    out_ref[...] = jax.lax.dot_general(
        hm_aug, w2_ref[...], (((0,), (0,)), ((), ())),
        preferred_element_type=jnp.float32,
    )


def kernel(species, embedding, idx_0, idx_1, idx_2, idx_3, idx_4, idx_5,
           idx_6, idx_7, W1, b1, W2, b2):
    idx_cat = jnp.concatenate(
        [idx_0, idx_1, idx_2, idx_3, idx_4, idx_5, idx_6, idx_7])
    idx3 = idx_cat.astype(jnp.int32).reshape(SC_NS, SC_K, 128)
    eid3 = jnp.broadcast_to(
        jnp.arange(E, dtype=jnp.int32)[:, None], (E, PER)
    ).reshape(SC_NS, SC_K, 128)

    eid = _sc_route_eid(idx3, eid3).reshape(1, N)

    w1_cat = jnp.transpose(W1, (1, 0, 2)).reshape(D, E * H).astype(jnp.bfloat16)
    b1_cat = b1.reshape(E * H, 1)
    w2_aug = jnp.concatenate(
        [W2.reshape(E * H, O), b2], axis=0).astype(jnp.bfloat16)

    out = pl.pallas_call(
        _mlp_kernel,
        grid=(N // TILE,),
        in_specs=[
            pl.BlockSpec((TILE, D), lambda i: (i, 0)),
            pl.BlockSpec((1, TILE), lambda i: (0, i)),
            pl.BlockSpec((D, E * H), lambda i: (0, 0)),
            pl.BlockSpec((E * H, 1), lambda i: (0, 0)),
            pl.BlockSpec((E * H + E, O), lambda i: (0, 0)),
        ],
        out_specs=pl.BlockSpec((TILE, O), lambda i: (i, 0)),
        out_shape=jax.ShapeDtypeStruct((N, O), jnp.float32),
        compiler_params=pltpu.CompilerParams(
            dimension_semantics=("arbitrary",),
        ),
    )(embedding, eid, w1_cat, b1_cat, w2_aug)
    return out
